# baseline (speedup 1.0000x reference)
import functools

import jax
import jax.numpy as jnp
from jax import lax
from jax.experimental import pallas as pl
from jax.experimental.pallas import tpu as pltpu
from jax.experimental.pallas import tpu_sc as plsc

NE = 8
HID = 1024
MI = 512
SI = 1024
TOKENS = 2048
TM = 256
NPAIR = 2 * TOKENS
NT = 24
ROWS = NT * TM
HW = HID // 2
NWORK = 32
CHUNK = NPAIR // NWORK


def _fdot(a, b):
    return lax.dot_general(a, b, (((1,), (1,)), ((), ())),
                           preferred_element_type=jnp.float32)


def _router_body(x_ref, rw_ref, eps_ref, wps_ref):
    x32 = x_ref[...]
    logits = lax.dot_general(x32, rw_ref[...], (((1,), (1,)), ((), ())),
                             preferred_element_type=jnp.float32)
    probs = jax.nn.softmax(logits, axis=-1)
    iota8 = lax.broadcasted_iota(jnp.int32, (TM, NE), 1)
    v1 = jnp.max(probs, axis=1, keepdims=True)
    i1 = jnp.min(jnp.where(probs >= v1, iota8, NE), axis=1, keepdims=True)
    pm = jnp.where(iota8 == i1, -1.0, probs)
    v2 = jnp.max(pm, axis=1, keepdims=True)
    i2 = jnp.min(jnp.where(pm >= v2, iota8, NE), axis=1, keepdims=True)
    rs = v1 + v2
    eps_ref[...] = jnp.concatenate([i1, i2], axis=1)
    wps_ref[...] = jnp.concatenate([v1 / rs, v2 / rs], axis=1)


@jax.jit
def _router(x, rw):
    return pl.pallas_call(
        _router_body,
        grid=(TOKENS // TM,),
        in_specs=[pl.BlockSpec((TM, HID), lambda t: (t, 0)),
                  pl.BlockSpec((NE, HID), lambda t: (0, 0))],
        out_specs=[pl.BlockSpec((TM, 2), lambda t: (t, 0)),
                   pl.BlockSpec((TM, 2), lambda t: (t, 0))],
        out_shape=[jax.ShapeDtypeStruct((TOKENS, 2), jnp.int32),
                   jax.ShapeDtypeStruct((TOKENS, 2), jnp.float32)],
    )(x, rw)


def _shared_body(x_ref, sg_ref, su_ref, sd_ref, seg_ref, out_ref):
    x32 = x_ref[...]
    xb = x32.astype(jnp.bfloat16)
    g = _fdot(xb, sg_ref[...])
    u = _fdot(xb, su_ref[...])
    hs = (jax.nn.silu(g) * u).astype(jnp.bfloat16)
    sy = _fdot(hs, sd_ref[...])
    gate = jax.nn.sigmoid(lax.dot_general(
        x32, seg_ref[...], (((1,), (1,)), ((), ())),
        preferred_element_type=jnp.float32,
        precision=lax.Precision.HIGHEST))
    out_ref[...] = (gate * sy).astype(jnp.bfloat16)


@jax.jit
def _shared(x, sg_b, su_b, sd_b, seg):
    full = lambda shape: pl.BlockSpec(shape, lambda t: tuple(0 for _ in shape))
    return pl.pallas_call(
        _shared_body,
        grid=(TOKENS // TM,),
        in_specs=[pl.BlockSpec((TM, HID), lambda t: (t, 0)),
                  full((SI, HID)), full((SI, HID)), full((HID, SI)),
                  full((1, HID))],
        out_specs=pl.BlockSpec((TM, HID), lambda t: (t, 0)),
        out_shape=jax.ShapeDtypeStruct((TOKENS, HID), jnp.bfloat16),
    )(x, sg_b, su_b, sd_b, seg)


def _lane_extract(vec, lane):
    li = jnp.arange(16, dtype=jnp.int32)
    return jnp.sum(jnp.where(li == lane, vec, 0), axis=0)


def _dispatch_body(ep_hbm, xb_hbm, xs_hbm, dest_hbm, te_hbm,
                   ep_cnt, ep_m, cnt_all, cnt_row, dest_v, tok_v, te_v,
                   rows_v, cnt_sh):
    c = lax.axis_index("c")
    s = lax.axis_index("s")
    wid = s * 2 + c
    li = jnp.arange(16, dtype=jnp.int32)
    zeros = jnp.zeros((16,), jnp.int32)

    for half in range(2):
        blk = 2 * s + half
        pltpu.sync_copy(ep_hbm.at[pl.ds(blk * CHUNK, CHUNK)], ep_cnt)

        ones = jnp.full((16,), 1, jnp.int32)

        def _cbody(g, cnt):
            vg = ep_cnt[pl.ds(16 * g, 16)]
            for k in range(16):
                ek = jnp.broadcast_to(vg[k], (16,))
                cnt = cnt + jnp.where(li == ek, ones, zeros)
            return cnt

        cnt_row[...] = lax.fori_loop(0, CHUNK // 16, _cbody, zeros)
        pltpu.sync_copy(cnt_row, cnt_sh.at[pl.ds(blk * 16, 16)])
    plsc.subcore_barrier()

    pltpu.sync_copy(cnt_sh, cnt_all)
    tot = zeros
    pre = zeros
    for b in range(NWORK):
        row = cnt_all[pl.ds(16 * b, 16)]
        tot = tot + row
        sel = jnp.broadcast_to((jnp.int32(b) < wid).astype(jnp.int32), (16,))
        pre = pre + row * sel
    nt = lax.shift_right_logical(tot + (TM - 1), jnp.full((16,), 8, jnp.int32))

    base = []
    end_tiles = []
    acc = jnp.int32(0)
    for e in range(NE):
        base.append(acc * TM + pre[e])
        acc = acc + nt[e]
        end_tiles.append(acc)

    te_lo = zeros
    te_hi = zeros
    ones = jnp.full((16,), 1, jnp.int32)
    for e in range(NE):
        end_b = jnp.broadcast_to(end_tiles[e], (16,))
        te_lo = te_lo + jnp.where(end_b <= li, ones, zeros)
        te_hi = te_hi + jnp.where(end_b <= li + 16, ones, zeros)
    te_v[pl.ds(0, 16)] = te_lo
    te_v[pl.ds(16, 16)] = te_hi

    @pl.when(wid == 0)
    def _():
        pltpu.sync_copy(te_v, te_hbm)

    pltpu.sync_copy(ep_hbm.at[pl.ds(wid * CHUNK, CHUNK)], ep_m)

    def _rbody(g, runs):
        vg = ep_m[pl.ds(16 * g, 16)]
        d = zeros
        new = list(runs)
        for k in range(16):
            ek = vg[k]
            dk = jnp.int32(0)
            for e in range(NE):
                msk = (ek == e).astype(jnp.int32)
                dk = dk + msk * new[e]
                new[e] = new[e] + msk
            d = d + jnp.where(li == k, jnp.broadcast_to(dk, (16,)), zeros)
        dest_v[pl.ds(16 * g, 16)] = d
        pbase = jnp.broadcast_to(wid * CHUNK + 16 * g, (16,))
        tok_v[pl.ds(16 * g, 16)] = lax.shift_right_logical(
            pbase + li, jnp.full((16,), 1, jnp.int32))
        return tuple(new)

    lax.fori_loop(0, CHUNK // 16, _rbody, tuple(base))
    pltpu.sync_copy(dest_v, dest_hbm.at[pl.ds(wid * CHUNK, CHUNK)])

    pltpu.sync_copy(xb_hbm.at[tok_v], rows_v)
    pltpu.sync_copy(rows_v, xs_hbm.at[dest_v])


_dispatch = pl.kernel(
    _dispatch_body,
    out_type=[jax.ShapeDtypeStruct((ROWS, HW), jnp.int32),
              jax.ShapeDtypeStruct((NPAIR,), jnp.int32),
              jax.ShapeDtypeStruct((NWORK,), jnp.int32)],
    mesh=plsc.VectorSubcoreMesh(core_axis_name="c", subcore_axis_name="s"),
    scratch_types=[
        pltpu.VMEM((CHUNK,), jnp.int32),
        pltpu.VMEM((CHUNK,), jnp.int32),
        pltpu.VMEM((NWORK * 16,), jnp.int32),
        pltpu.VMEM((16,), jnp.int32),
        pltpu.VMEM((CHUNK,), jnp.int32),
        pltpu.VMEM((CHUNK,), jnp.int32),
        pltpu.VMEM((NWORK,), jnp.int32),
        pltpu.VMEM((CHUNK, HW), jnp.int32),
        pltpu.VMEM_SHARED((NWORK * 16,), jnp.int32),
    ],
)


def _experts_body(te_ref, xs_ref, gu_ref, dn_ref, ys_ref):
    e = te_ref[pl.program_id(0)]

    @pl.when(e < NE)
    def _():
        xb = xs_ref[...]
        gu = _fdot(xb, gu_ref[0])
        h = (jax.nn.silu(gu[:, :MI]) * gu[:, MI:]).astype(jnp.bfloat16)
        ys_ref[...] = _fdot(h, dn_ref[0]).astype(jnp.bfloat16)


@jax.jit
def _experts(te, xs_bf, gu_b, dn_b):
    grid_spec = pltpu.PrefetchScalarGridSpec(
        num_scalar_prefetch=1,
        grid=(NT,),
        in_specs=[
            pl.BlockSpec((TM, HID), lambda i, te: (i, 0)),
            pl.BlockSpec((1, 2 * MI, HID),
                         lambda i, te: (jnp.minimum(te[i], NE - 1), 0, 0)),
            pl.BlockSpec((1, HID, MI),
                         lambda i, te: (jnp.minimum(te[i], NE - 1), 0, 0)),
        ],
        out_specs=pl.BlockSpec((TM, HID), lambda i, te: (i, 0)),
    )
    return pl.pallas_call(
        _experts_body,
        grid_spec=grid_spec,
        out_shape=jax.ShapeDtypeStruct((ROWS, HID), jnp.bfloat16),
    )(te, xs_bf, gu_b, dn_b)


def _unperm_body(dest_hbm, ys_hbm, yp_hbm, idx_v, rows_v):
    c = lax.axis_index("c")
    s = lax.axis_index("s")
    wid = s * 2 + c
    pltpu.sync_copy(dest_hbm.at[pl.ds(wid * CHUNK, CHUNK)], idx_v)
    pltpu.sync_copy(ys_hbm.at[idx_v], rows_v)
    pltpu.sync_copy(rows_v, yp_hbm.at[pl.ds(wid * CHUNK, CHUNK)])


_unpermute = pl.kernel(
    _unperm_body,
    out_type=jax.ShapeDtypeStruct((NPAIR, HW), jnp.int32),
    mesh=plsc.VectorSubcoreMesh(core_axis_name="c", subcore_axis_name="s"),
    scratch_types=[
        pltpu.VMEM((CHUNK,), jnp.int32),
        pltpu.VMEM((CHUNK, HW), jnp.int32),
    ],
)


def _combine_body(sh_ref, wps_ref, yp_ref, out_ref):
    y = yp_ref[...].astype(jnp.float32)
    w = wps_ref[...]
    out_ref[...] = (sh_ref[...].astype(jnp.float32)
                    + w[:, 0:1] * y[:, 0, :] + w[:, 1:2] * y[:, 1, :])


@jax.jit
def _combine(shared_b, wps, yp32):
    return pl.pallas_call(
        _combine_body,
        grid=(TOKENS // TM,),
        in_specs=[pl.BlockSpec((TM, HID), lambda t: (t, 0)),
                  pl.BlockSpec((TM, 2), lambda t: (t, 0)),
                  pl.BlockSpec((TM, 2, HID), lambda t: (t, 0, 0))],
        out_specs=pl.BlockSpec((TM, HID), lambda t: (t, 0)),
        out_shape=jax.ShapeDtypeStruct((TOKENS, HID), jnp.float32),
    )(shared_b, wps, yp32)


def kernel(hidden_states, router_weight, gate_up_proj, down_proj,
           shared_gate_proj, shared_up_proj, shared_down_proj,
           shared_expert_gate):
    B, S, H = hidden_states.shape
    x = hidden_states.reshape(-1, H)
    xb = x.astype(jnp.bfloat16)
    xb32 = lax.bitcast_convert_type(xb.reshape(TOKENS, HW, 2), jnp.int32)

    eps, wps = _router(x, router_weight)
    shared_b = _shared(x,
                       shared_gate_proj.astype(jnp.bfloat16),
                       shared_up_proj.astype(jnp.bfloat16),
                       shared_down_proj.astype(jnp.bfloat16),
                       shared_expert_gate)
    xs32, dest, te = _dispatch(eps.reshape(NPAIR), xb32)
    xs_bf = lax.bitcast_convert_type(xs32, jnp.bfloat16).reshape(ROWS, HID)
    ys_bf = _experts(te, xs_bf,
                     gate_up_proj.astype(jnp.bfloat16),
                     down_proj.astype(jnp.bfloat16))
    ys32 = lax.bitcast_convert_type(ys_bf.reshape(ROWS, HW, 2), jnp.int32)
    yp32 = _unpermute(dest, ys32)
    yp_bf = lax.bitcast_convert_type(yp32, jnp.bfloat16)
    out = _combine(shared_b, wps, yp_bf.reshape(TOKENS, 2, HID))
    return out.reshape(B, S, H)


# speedup vs baseline: 8.7017x; 8.7017x over previous
import functools

import jax
import jax.numpy as jnp
from jax import lax
from jax.experimental import pallas as pl
from jax.experimental.pallas import tpu as pltpu
from jax.experimental.pallas import tpu_sc as plsc

NE = 8
HID = 1024
MI = 512
SI = 1024
TOKENS = 2048
TM = 256
NPAIR = 2 * TOKENS
NT = 24
ROWS = NT * TM
HW = HID // 2
NWORK = 32
CHUNK = NPAIR // NWORK


def _fdot(a, b):
    return lax.dot_general(a, b, (((1,), (1,)), ((), ())),
                           preferred_element_type=jnp.float32)


def _router_body(x_ref, rw_ref, eps_ref, wps_ref):
    x32 = x_ref[...]
    logits = lax.dot_general(x32, rw_ref[...], (((1,), (1,)), ((), ())),
                             preferred_element_type=jnp.float32)
    probs = jax.nn.softmax(logits, axis=-1)
    iota8 = lax.broadcasted_iota(jnp.int32, (TM, NE), 1)
    v1 = jnp.max(probs, axis=1, keepdims=True)
    i1 = jnp.min(jnp.where(probs >= v1, iota8, NE), axis=1, keepdims=True)
    pm = jnp.where(iota8 == i1, -1.0, probs)
    v2 = jnp.max(pm, axis=1, keepdims=True)
    i2 = jnp.min(jnp.where(pm >= v2, iota8, NE), axis=1, keepdims=True)
    rs = v1 + v2
    eps_ref[...] = jnp.concatenate([i1, i2], axis=1)
    wps_ref[...] = jnp.concatenate([v1 / rs, v2 / rs], axis=1)


@jax.jit
def _router(x, rw):
    return pl.pallas_call(
        _router_body,
        grid=(TOKENS // TM,),
        in_specs=[pl.BlockSpec((TM, HID), lambda t: (t, 0)),
                  pl.BlockSpec((NE, HID), lambda t: (0, 0))],
        out_specs=[pl.BlockSpec((TM, 2), lambda t: (t, 0)),
                   pl.BlockSpec((TM, 2), lambda t: (t, 0))],
        out_shape=[jax.ShapeDtypeStruct((TOKENS, 2), jnp.int32),
                   jax.ShapeDtypeStruct((TOKENS, 2), jnp.float32)],
    )(x, rw)


def _shared_body(x_ref, sg_ref, su_ref, sd_ref, seg_ref, out_ref):
    x32 = x_ref[...]
    xb = x32.astype(jnp.bfloat16)
    g = _fdot(xb, sg_ref[...])
    u = _fdot(xb, su_ref[...])
    hs = (jax.nn.silu(g) * u).astype(jnp.bfloat16)
    sy = _fdot(hs, sd_ref[...])
    gate = jax.nn.sigmoid(lax.dot_general(
        x32, seg_ref[...], (((1,), (1,)), ((), ())),
        preferred_element_type=jnp.float32,
        precision=lax.Precision.HIGHEST))
    out_ref[...] = gate * sy


@jax.jit
def _shared(x, sg_b, su_b, sd_b, seg):
    full = lambda shape: pl.BlockSpec(shape, lambda t: tuple(0 for _ in shape))
    return pl.pallas_call(
        _shared_body,
        grid=(TOKENS // TM,),
        in_specs=[pl.BlockSpec((TM, HID), lambda t: (t, 0)),
                  full((SI, HID)), full((SI, HID)), full((HID, SI)),
                  full((1, HID))],
        out_specs=pl.BlockSpec((TM, HID), lambda t: (t, 0)),
        out_shape=jax.ShapeDtypeStruct((TOKENS, HID), jnp.float32),
    )(x, sg_b, su_b, sd_b, seg)


def _lane_extract(vec, lane):
    li = jnp.arange(16, dtype=jnp.int32)
    return jnp.sum(jnp.where(li == lane, vec, 0), axis=0)


def _dispatch_body(ep_hbm, x_hbm, xs_hbm, dest_hbm, te_hbm,
                   ep_cnt, ep_m, cnt_all, cnt_row, dest_a, dest_b,
                   tok_a, tok_b, te_v, rows_v, cnt_sh):
    c = lax.axis_index("c")
    s = lax.axis_index("s")
    wid = s * 2 + c
    li = jnp.arange(16, dtype=jnp.int32)
    zeros = jnp.zeros((16,), jnp.int32)

    for half in range(2):
        blk = 2 * s + half
        pltpu.sync_copy(ep_hbm.at[pl.ds(blk * CHUNK, CHUNK)], ep_cnt)

        ones = jnp.full((16,), 1, jnp.int32)

        def _cbody(g, cnt):
            vg = ep_cnt[pl.ds(16 * g, 16)]
            for k in range(16):
                ek = jnp.broadcast_to(vg[k], (16,))
                cnt = cnt + jnp.where(li == ek, ones, zeros)
            return cnt

        cnt_row[...] = lax.fori_loop(0, CHUNK // 16, _cbody, zeros)
        pltpu.sync_copy(cnt_row, cnt_sh.at[pl.ds(blk * 16, 16)])
    plsc.subcore_barrier()

    pltpu.sync_copy(cnt_sh, cnt_all)
    tot = zeros
    pre = zeros
    for b in range(NWORK):
        row = cnt_all[pl.ds(16 * b, 16)]
        tot = tot + row
        sel = jnp.broadcast_to((jnp.int32(b) < wid).astype(jnp.int32), (16,))
        pre = pre + row * sel
    nt = lax.shift_right_logical(tot + (TM - 1), jnp.full((16,), 8, jnp.int32))

    base = []
    end_tiles = []
    acc = jnp.int32(0)
    for e in range(NE):
        base.append(acc * TM + pre[e])
        acc = acc + nt[e]
        end_tiles.append(acc)

    te_lo = zeros
    te_hi = zeros
    ones = jnp.full((16,), 1, jnp.int32)
    for e in range(NE):
        end_b = jnp.broadcast_to(end_tiles[e], (16,))
        te_lo = te_lo + jnp.where(end_b <= li, ones, zeros)
        te_hi = te_hi + jnp.where(end_b <= li + 16, ones, zeros)
    te_v[pl.ds(0, 16)] = te_lo
    te_v[pl.ds(16, 16)] = te_hi

    @pl.when(wid == 0)
    def _():
        pltpu.sync_copy(te_v, te_hbm)

    pltpu.sync_copy(ep_hbm.at[pl.ds(wid * CHUNK, CHUNK)], ep_m)

    def _make_rbody(dest_ref, tok_ref, goff):
        def _rbody(g, runs):
            vg = ep_m[pl.ds(16 * g, 16)]
            d = zeros
            new = list(runs)
            for k in range(16):
                ek = vg[k]
                dk = jnp.int32(0)
                for e in range(NE):
                    msk = (ek == e).astype(jnp.int32)
                    dk = dk + msk * new[e]
                    new[e] = new[e] + msk
                d = d + jnp.where(li == k, jnp.broadcast_to(dk, (16,)),
                                  zeros)
            dest_ref[pl.ds(16 * g - goff, 16)] = d
            pbase = jnp.broadcast_to(wid * CHUNK + 16 * g, (16,))
            tok_ref[pl.ds(16 * g - goff, 16)] = lax.shift_right_logical(
                pbase + li, jnp.full((16,), 1, jnp.int32))
            return tuple(new)
        return _rbody

    runs = lax.fori_loop(0, 4, _make_rbody(dest_a, tok_a, 0), tuple(base))
    lax.fori_loop(4, 8, _make_rbody(dest_b, tok_b, 64), runs)
    pltpu.sync_copy(dest_a, dest_hbm.at[pl.ds(wid * CHUNK, 64)])
    pltpu.sync_copy(dest_b, dest_hbm.at[pl.ds(wid * CHUNK + 64, 64)])

    pltpu.sync_copy(x_hbm.at[tok_a], rows_v)
    pltpu.sync_copy(rows_v, xs_hbm.at[dest_a])
    pltpu.sync_copy(x_hbm.at[tok_b], rows_v)
    pltpu.sync_copy(rows_v, xs_hbm.at[dest_b])


_dispatch = pl.kernel(
    _dispatch_body,
    out_type=[jax.ShapeDtypeStruct((ROWS, HID), jnp.float32),
              jax.ShapeDtypeStruct((NPAIR,), jnp.int32),
              jax.ShapeDtypeStruct((NWORK,), jnp.int32)],
    mesh=plsc.VectorSubcoreMesh(core_axis_name="c", subcore_axis_name="s"),
    scratch_types=[
        pltpu.VMEM((CHUNK,), jnp.int32),
        pltpu.VMEM((CHUNK,), jnp.int32),
        pltpu.VMEM((NWORK * 16,), jnp.int32),
        pltpu.VMEM((16,), jnp.int32),
        pltpu.VMEM((64,), jnp.int32),
        pltpu.VMEM((64,), jnp.int32),
        pltpu.VMEM((64,), jnp.int32),
        pltpu.VMEM((64,), jnp.int32),
        pltpu.VMEM((NWORK,), jnp.int32),
        pltpu.VMEM((64, HID), jnp.float32),
        pltpu.VMEM_SHARED((NWORK * 16,), jnp.int32),
    ],
)


def _experts_body(te_ref, xs_ref, gu_ref, dn_ref, ys_ref):
    e = te_ref[pl.program_id(0)]

    @pl.when(e < NE)
    def _():
        xb = xs_ref[...].astype(jnp.bfloat16)
        gu = _fdot(xb, gu_ref[0])
        h = (jax.nn.silu(gu[:, :MI]) * gu[:, MI:]).astype(jnp.bfloat16)
        ys_ref[...] = _fdot(h, dn_ref[0])


@jax.jit
def _experts(te, xs_bf, gu_b, dn_b):
    grid_spec = pltpu.PrefetchScalarGridSpec(
        num_scalar_prefetch=1,
        grid=(NT,),
        in_specs=[
            pl.BlockSpec((TM, HID), lambda i, te: (i, 0)),
            pl.BlockSpec((1, 2 * MI, HID),
                         lambda i, te: (jnp.minimum(te[i], NE - 1), 0, 0)),
            pl.BlockSpec((1, HID, MI),
                         lambda i, te: (jnp.minimum(te[i], NE - 1), 0, 0)),
        ],
        out_specs=pl.BlockSpec((TM, HID), lambda i, te: (i, 0)),
    )
    return pl.pallas_call(
        _experts_body,
        grid_spec=grid_spec,
        out_shape=jax.ShapeDtypeStruct((ROWS, HID), jnp.float32),
    )(te, xs_bf, gu_b, dn_b)


def _unperm_body(dest_hbm, ys_hbm, yp_hbm, idx_v, rows_v):
    c = lax.axis_index("c")
    s = lax.axis_index("s")
    wid = s * 2 + c
    for h in range(2):
        off = wid * CHUNK + 64 * h
        pltpu.sync_copy(dest_hbm.at[pl.ds(off, 64)], idx_v)
        pltpu.sync_copy(ys_hbm.at[idx_v], rows_v)
        pltpu.sync_copy(rows_v, yp_hbm.at[pl.ds(off, 64)])


_unpermute = pl.kernel(
    _unperm_body,
    out_type=jax.ShapeDtypeStruct((NPAIR, HID), jnp.float32),
    mesh=plsc.VectorSubcoreMesh(core_axis_name="c", subcore_axis_name="s"),
    scratch_types=[
        pltpu.VMEM((64,), jnp.int32),
        pltpu.VMEM((64, HID), jnp.float32),
    ],
)


def _combine_body(sh_ref, wps_ref, yp_ref, out_ref):
    y = yp_ref[...].astype(jnp.float32)
    w = wps_ref[...]
    out_ref[...] = (sh_ref[...].astype(jnp.float32)
                    + w[:, 0:1] * y[:, 0, :] + w[:, 1:2] * y[:, 1, :])


@jax.jit
def _combine(shared_b, wps, yp32):
    return pl.pallas_call(
        _combine_body,
        grid=(TOKENS // TM,),
        in_specs=[pl.BlockSpec((TM, HID), lambda t: (t, 0)),
                  pl.BlockSpec((TM, 2), lambda t: (t, 0)),
                  pl.BlockSpec((TM, 2, HID), lambda t: (t, 0, 0))],
        out_specs=pl.BlockSpec((TM, HID), lambda t: (t, 0)),
        out_shape=jax.ShapeDtypeStruct((TOKENS, HID), jnp.float32),
    )(shared_b, wps, yp32)


def kernel(hidden_states, router_weight, gate_up_proj, down_proj,
           shared_gate_proj, shared_up_proj, shared_down_proj,
           shared_expert_gate):
    B, S, H = hidden_states.shape
    x = hidden_states.reshape(-1, H)

    eps, wps = _router(x, router_weight)
    shared_o = _shared(x,
                       shared_gate_proj.astype(jnp.bfloat16),
                       shared_up_proj.astype(jnp.bfloat16),
                       shared_down_proj.astype(jnp.bfloat16),
                       shared_expert_gate)
    xs, dest, te = _dispatch(eps.reshape(NPAIR), x)
    ys = _experts(te, xs,
                  gate_up_proj.astype(jnp.bfloat16),
                  down_proj.astype(jnp.bfloat16))
    yp = _unpermute(dest, ys)
    out = _combine(shared_o, wps, yp.reshape(TOKENS, 2, HID))
    return out.reshape(B, S, H)
